# baseline (speedup 1.0000x reference)
import functools

import jax
import jax.numpy as jnp
import numpy as np
from jax.experimental import pallas as pl

_TARGET_RATE = 0.5
_BLOCK_COLS = 256


def _select_mask_kernel(k, x_ref, o_ref):
    x = x_ref[...]
    u = jax.lax.bitcast_convert_type(jnp.abs(x), jnp.int32)
    rows = x.shape[0]
    prefix = jnp.zeros((1, x.shape[1]), dtype=jnp.int32)
    for b in range(30, -1, -1):
        cand = prefix | ((1 << b) - 1)
        cnt = jnp.sum((u <= cand).astype(jnp.int32), axis=0, keepdims=True)
        prefix = jnp.where(cnt >= k, prefix, prefix | (1 << b))
    o_ref[...] = jnp.where(u <= prefix, jnp.zeros_like(x), x)


def _run_block(x2, k):
    rows, d = x2.shape
    bc = min(_BLOCK_COLS, d)
    return pl.pallas_call(
        functools.partial(_select_mask_kernel, k),
        grid=(d // bc,),
        in_specs=[pl.BlockSpec((rows, bc), lambda j: (0, j))],
        out_specs=pl.BlockSpec((rows, bc), lambda j: (0, j)),
        out_shape=jax.ShapeDtypeStruct((rows, d), x2.dtype),
    )(x2)


def kernel(inputs):
    shape = inputs.shape
    d = shape[-1]
    rows = 1
    for s in shape[:-1]:
        rows *= s
    k = int(_TARGET_RATE * float(rows))
    x2 = inputs.reshape(rows, d)
    devs = jax.devices()
    nd = 2 if (len(devs) >= 2 and d % (2 * _BLOCK_COLS) == 0) else 1
    if nd > 1:
        mesh = jax.sharding.Mesh(np.array(devs[:nd]), ("c",))
        p = jax.sharding.PartitionSpec
        fn = jax.shard_map(
            functools.partial(_run_block, k=k),
            mesh=mesh,
            in_specs=p(None, "c"),
            out_specs=p(None, "c"),
            check_vma=False,
        )
        out = fn(x2)
    else:
        out = _run_block(x2, k)
    return out.reshape(shape)


# speedup vs baseline: 2.5833x; 2.5833x over previous
import functools

import jax
import jax.numpy as jnp
import numpy as np
from jax.experimental import pallas as pl

_TARGET_RATE = 0.5
_BLOCK_COLS = 256


def _count_i16(mask):
    acc = mask.astype(jnp.int16)
    r = acc.shape[0]
    while r > 16:
        r //= 2
        acc = acc[:r] + acc[r:]
    s = jnp.sum(acc.astype(jnp.int32), axis=0, keepdims=True)
    return s.astype(jnp.int16)


def _select_mask_kernel(k, x_ref, o_ref):
    x = x_ref[...]
    u = jax.lax.bitcast_convert_type(jnp.abs(x), jnp.int32)
    i16 = jnp.int16

    h = ((u >> 15) - 32768).astype(i16)
    prefix = jnp.full((1, x.shape[1]), -32768, dtype=i16)
    for b in range(15, -1, -1):
        bit = i16(-32768) if b == 15 else i16(1 << b)
        low = i16((1 << b) - 1)
        cand = prefix | low
        cnt = _count_i16(h <= cand)
        prefix = jnp.where(cnt >= i16(k), prefix, prefix ^ bit)
    hp = prefix

    base = _count_i16(h < hp)
    m = h == hp
    k2 = i16(k) - base

    lo = ((u & 0x7FFF) - 32768).astype(i16)
    lm = jnp.where(m, lo, i16(0))
    prefix2 = jnp.full((1, x.shape[1]), -32768, dtype=i16)
    for b in range(14, -1, -1):
        cand = prefix2 | i16((1 << b) - 1)
        cnt = _count_i16(lm <= cand)
        prefix2 = jnp.where(cnt >= k2, prefix2, prefix2 | i16(1 << b))

    v = ((hp.astype(jnp.int32) + 32768) << 15) | (
        prefix2.astype(jnp.int32) + 32768)
    o_ref[...] = jnp.where(u <= v, jnp.zeros_like(x), x)


def _run_block(x2, k):
    rows, d = x2.shape
    bc = min(_BLOCK_COLS, d)
    return pl.pallas_call(
        functools.partial(_select_mask_kernel, k),
        grid=(d // bc,),
        in_specs=[pl.BlockSpec((rows, bc), lambda j: (0, j))],
        out_specs=pl.BlockSpec((rows, bc), lambda j: (0, j)),
        out_shape=jax.ShapeDtypeStruct((rows, d), x2.dtype),
    )(x2)


def kernel(inputs):
    shape = inputs.shape
    d = shape[-1]
    rows = 1
    for s in shape[:-1]:
        rows *= s
    k = int(_TARGET_RATE * float(rows))
    x2 = inputs.reshape(rows, d)
    out = _run_block(x2, k)
    return out.reshape(shape)
